# Initial kernel scaffold; baseline (speedup 1.0000x reference)
import jax
import jax.numpy as jnp
from jax.experimental import pallas as pl


def kernel(x, edge_index, batch, W1, b1, g1, be1, W2, b2, g2, be2, W3, b3, g3, be3, ln_g, ln_b, att, mw1, mb1, mw2, mb2, mw3, mb3):
    raise NotImplementedError("write your pallas kernel here")


# speedup vs baseline: 9.0810x; 9.0810x over previous
import functools

import jax
import jax.numpy as jnp
from jax import lax
from jax.experimental import pallas as pl
from jax.experimental.pallas import tpu as pltpu
from jax.experimental.pallas import tpu_sc as plsc

_N = 10000
_E = 320000
_B = 64
_NC = 2
_NS = 16
_NW = _NC * _NS
_NPAD = 10240
_RPS = _NPAD // _NS
_EPW = _E // _NW
_K = 80
_NCH = _EPW // _K
_RB = 2000
_GRID = _N // _RB
_EPS = 1e-5

_MESH = dict(core_axis_name="c", subcore_axis_name="s",
             num_cores=_NC, num_subcores=_NS)


def _make_sc_degree():
    @functools.partial(
        pl.kernel,
        out_type=jax.ShapeDtypeStruct((_NC, _NPAD, 128), jnp.float32),
        mesh=plsc.VectorSubcoreMesh(**_MESH),
        scratch_types=[
            pltpu.VMEM((_K,), jnp.int32),
            pltpu.VMEM((_K, 128), jnp.float32),
            pltpu.VMEM_SHARED((_NPAD, 128), jnp.float32),
        ],
    )
    def deg_kernel(dst_hbm, ones_hbm, zero_hbm, out_hbm, idx_v, ones_v, acc_sh):
        c = lax.axis_index("c")
        s = lax.axis_index("s")
        wid = s * _NC + c
        r0 = s * _RPS
        pltpu.sync_copy(zero_hbm, acc_sh.at[pl.ds(r0, _RPS)])
        pltpu.sync_copy(ones_hbm, ones_v)
        plsc.subcore_barrier()
        base = wid * _EPW

        def step(i, carry):
            off = pl.multiple_of(base + i * _K, 8)
            pltpu.sync_copy(dst_hbm.at[pl.ds(off, _K)], idx_v)
            pltpu.sync_copy(ones_v, acc_sh.at[idx_v], add=True)
            return carry

        lax.fori_loop(0, _NCH, step, 0)
        plsc.subcore_barrier()
        pltpu.sync_copy(acc_sh.at[pl.ds(r0, _RPS)],
                        out_hbm.at[c, pl.ds(r0, _RPS)])

    return deg_kernel


def _make_sc_segsum(fc, p):
    @functools.partial(
        pl.kernel,
        out_type=jax.ShapeDtypeStruct((p * _NC, _NPAD, fc), jnp.float32),
        mesh=plsc.VectorSubcoreMesh(**_MESH),
        scratch_types=[
            pltpu.VMEM((_K,), jnp.int32),
            pltpu.VMEM((_K,), jnp.int32),
            pltpu.VMEM((_K, fc), jnp.float32),
            pltpu.VMEM_SHARED((_NPAD, fc), jnp.float32),
            pltpu.SemaphoreType.DMA,
        ],
    )
    def seg_kernel(*refs):
        hps = refs[:p]
        src_hbm, dst_hbm, zero_hbm, out_hbm = refs[p:p + 4]
        sidx, didx, rows_v, acc_sh, sem = refs[p + 4:]
        c = lax.axis_index("c")
        s = lax.axis_index("s")
        wid = s * _NC + c
        r0 = s * _RPS
        base = wid * _EPW
        for pp in range(p):
            hp = hps[pp]
            pltpu.sync_copy(zero_hbm, acc_sh.at[pl.ds(r0, _RPS)])
            plsc.subcore_barrier()

            def step(i, carry):
                off = pl.multiple_of(base + i * _K, 8)
                pltpu.sync_copy(src_hbm.at[pl.ds(off, _K)], sidx)
                pltpu.sync_copy(dst_hbm.at[pl.ds(off, _K)], didx)
                pltpu.async_copy(hp.at[sidx], rows_v, sem).wait()
                pltpu.sync_copy(rows_v, acc_sh.at[didx], add=True)
                return carry

            lax.fori_loop(0, _NCH, step, 0)
            plsc.subcore_barrier()
            pltpu.sync_copy(acc_sh.at[pl.ds(r0, _RPS)],
                            out_hbm.at[pp * _NC + c, pl.ds(r0, _RPS)])
            if pp + 1 < p:
                plsc.subcore_barrier()

    return seg_kernel


def _tc_mm1(x, w1, degp):
    def body(x_ref, w_ref, deg_ref, hlo_ref, hhi_ref, dinv_ref):
        deg = 1.0 + deg_ref[0][:, 0:1] + deg_ref[1][:, 0:1]
        dinv = lax.rsqrt(deg)
        h = jnp.dot(x_ref[...], w_ref[...], preferred_element_type=jnp.float32)
        hp = h * dinv
        hlo_ref[...] = hp[:, :128]
        hhi_ref[...] = hp[:, 128:]
        dinv_ref[...] = dinv

    return pl.pallas_call(
        body,
        grid=(_GRID,),
        in_specs=[
            pl.BlockSpec((_RB, 128), lambda i: (i, 0)),
            pl.BlockSpec((128, 256), lambda i: (0, 0)),
            pl.BlockSpec((_NC, _RB, 128), lambda i: (0, i, 0)),
        ],
        out_specs=[
            pl.BlockSpec((_RB, 128), lambda i: (i, 0)),
            pl.BlockSpec((_RB, 128), lambda i: (i, 0)),
            pl.BlockSpec((_RB, 1), lambda i: (i, 0)),
        ],
        out_shape=[
            jax.ShapeDtypeStruct((_N, 128), jnp.float32),
            jax.ShapeDtypeStruct((_N, 128), jnp.float32),
            jax.ShapeDtypeStruct((_N, 1), jnp.float32),
        ],
    )(x, w1, degp)


def _tc_stats(parts, hps, dinv, b, f, p, wc=128):
    fc = f // p

    def body(*refs):
        parts_ref = refs[0]
        hp_refs = refs[1:1 + p]
        dinv_ref, b_ref = refs[1 + p:3 + p]
        op_ref, mu_ref, var_ref = refs[3 + p:6 + p]
        s_acc, ss_acc = refs[6 + p:]
        i = pl.program_id(0)
        dinv = dinv_ref[...]
        cols = []
        for pp in range(p):
            tot = parts_ref[2 * pp] + parts_ref[2 * pp + 1] + hp_refs[pp][...]
            cols.append(dinv * tot[:, :fc])
        outb = (jnp.concatenate(cols, axis=1) if p > 1 else cols[0]) + b_ref[...]
        op_ref[...] = outb

        @pl.when(i == 0)
        def _():
            s_acc[...] = jnp.zeros_like(s_acc)
            ss_acc[...] = jnp.zeros_like(ss_acc)

        s_acc[...] += jnp.sum(outb, axis=0, keepdims=True)
        ss_acc[...] += jnp.sum(outb * outb, axis=0, keepdims=True)

        @pl.when(i == _GRID - 1)
        def _():
            mu = s_acc[...] * (1.0 / _N)
            mu_ref[...] = mu
            var_ref[...] = ss_acc[...] * (1.0 / _N) - mu * mu

    return pl.pallas_call(
        body,
        grid=(_GRID,),
        in_specs=(
            [pl.BlockSpec((p * _NC, _RB, wc), lambda i: (0, i, 0))]
            + [pl.BlockSpec((_RB, wc), lambda i: (i, 0))] * p
            + [pl.BlockSpec((_RB, 1), lambda i: (i, 0)),
               pl.BlockSpec((1, f), lambda i: (0, 0))]
        ),
        out_specs=[
            pl.BlockSpec((_RB, f), lambda i: (i, 0)),
            pl.BlockSpec((1, f), lambda i: (0, 0)),
            pl.BlockSpec((1, f), lambda i: (0, 0)),
        ],
        out_shape=[
            jax.ShapeDtypeStruct((_N, f), jnp.float32),
            jax.ShapeDtypeStruct((1, f), jnp.float32),
            jax.ShapeDtypeStruct((1, f), jnp.float32),
        ],
        scratch_shapes=[
            pltpu.VMEM((1, f), jnp.float32),
            pltpu.VMEM((1, f), jnp.float32),
        ],
    )(parts, *hps, dinv, b)


def _tc_bnmm(op, mu, var, g, be, w, dinv, f, fn):
    def body(op_ref, mu_ref, var_ref, g_ref, be_ref, w_ref, dinv_ref, out_ref):
        hn = (op_ref[...] - mu_ref[...]) * lax.rsqrt(var_ref[...] + _EPS)
        hn = jnp.maximum(hn * g_ref[...] + be_ref[...], 0.0)
        out_ref[...] = jnp.dot(
            hn, w_ref[...], preferred_element_type=jnp.float32) * dinv_ref[...]

    return pl.pallas_call(
        body,
        grid=(_GRID,),
        in_specs=[
            pl.BlockSpec((_RB, f), lambda i: (i, 0)),
            pl.BlockSpec((1, f), lambda i: (0, 0)),
            pl.BlockSpec((1, f), lambda i: (0, 0)),
            pl.BlockSpec((1, f), lambda i: (0, 0)),
            pl.BlockSpec((1, f), lambda i: (0, 0)),
            pl.BlockSpec((f, fn), lambda i: (0, 0)),
            pl.BlockSpec((_RB, 1), lambda i: (i, 0)),
        ],
        out_specs=pl.BlockSpec((_RB, fn), lambda i: (i, 0)),
        out_shape=jax.ShapeDtypeStruct((_N, fn), jnp.float32),
    )(op, mu, var, g, be, w, dinv)


def _tc_tail(op3, mu3, var3, g3, be3, lng, lnb, att, batch2d,
             mw1, mb1, mw2, mb2, mw3, mb3):
    def body(op_ref, mu_ref, var_ref, g_ref, be_ref, lg_ref, lb_ref, att_ref,
             bt_ref, w1_ref, c1_ref, w2_ref, c2_ref, w3_ref, c3_ref, out_ref):
        h = (op_ref[...] - mu_ref[...]) * lax.rsqrt(var_ref[...] + _EPS)
        h = jnp.maximum(h * g_ref[...] + be_ref[...], 0.0)
        rmu = jnp.mean(h, axis=1, keepdims=True)
        rvar = jnp.mean((h - rmu) ** 2, axis=1, keepdims=True)
        hln = (h - rmu) * lax.rsqrt(rvar + _EPS) * lg_ref[...] + lb_ref[...]
        t = jnp.tanh(jnp.dot(hln, att_ref[...],
                             preferred_element_type=jnp.float32))
        m = jnp.max(t, axis=0, keepdims=True)
        e = jnp.exp(t - m)
        aw = e / jnp.sum(e, axis=0, keepdims=True)
        hw = hln * aw
        ids = lax.broadcasted_iota(jnp.int32, (_B, 1), 0)
        oh = (bt_ref[...] == ids).astype(jnp.float32)
        sums = jnp.dot(oh, hw, preferred_element_type=jnp.float32)
        cnt = jnp.sum(oh, axis=1, keepdims=True)
        pooled = sums / jnp.maximum(cnt, 1.0)
        z = jnp.maximum(jnp.dot(pooled, w1_ref[...],
                                preferred_element_type=jnp.float32)
                        + c1_ref[...], 0.0)
        z = jnp.maximum(jnp.dot(z, w2_ref[...],
                                preferred_element_type=jnp.float32)
                        + c2_ref[...], 0.0)
        out_ref[...] = jnp.dot(z, w3_ref[...],
                               preferred_element_type=jnp.float32) + c3_ref[...]

    return pl.pallas_call(
        body,
        out_shape=jax.ShapeDtypeStruct((_B, 2), jnp.float32),
    )(op3, mu3, var3, g3, be3, lng, lnb, att, batch2d,
      mw1, mb1, mw2, mb2, mw3, mb3)


_deg_call = _make_sc_degree()
_seg128x2 = _make_sc_segsum(128, 2)
_seg128x1 = _make_sc_segsum(128, 1)


def kernel(x, edge_index, batch, W1, b1, g1, be1, W2, b2, g2, be2,
           W3, b3, g3, be3, ln_g, ln_b, att, mw1, mb1, mw2, mb2, mw3, mb3):
    src = edge_index[0]
    dst = edge_index[1]
    batch2d = batch.reshape(1, _N)
    ones128 = jnp.ones((_K, 128), jnp.float32)
    z128 = jnp.zeros((_RPS, 128), jnp.float32)
    w3p = jnp.concatenate([W3, jnp.zeros((128, 64), jnp.float32)], axis=1)
    row = lambda v: v.reshape(1, -1)

    degp = _deg_call(dst, ones128, z128)
    hlo, hhi, dinv = _tc_mm1(x, W1, degp)

    parts1 = _seg128x2(hlo, hhi, src, dst, z128)
    op1, mu1, var1 = _tc_stats(parts1, [hlo, hhi], dinv, row(b1), 256, 2)
    hp2 = _tc_bnmm(op1, mu1, var1, row(g1), row(be1), W2, dinv, 256, 128)

    parts2 = _seg128x1(hp2, src, dst, z128)
    op2, mu2, var2 = _tc_stats(parts2, [hp2], dinv, row(b2), 128, 1)
    hp3 = _tc_bnmm(op2, mu2, var2, row(g2), row(be2), w3p, dinv, 128, 128)

    parts3 = _seg128x1(hp3, src, dst, z128)
    op3, mu3, var3 = _tc_stats(parts3, [hp3], dinv, row(b3), 64, 1)

    return _tc_tail(op3, mu3, var3, row(g3), row(be3), row(ln_g), row(ln_b),
                    att, batch2d, mw1, row(mb1), mw2, row(mb2), mw3, row(mb3))


# speedup vs baseline: 14.6069x; 1.6085x over previous
import functools

import jax
import jax.numpy as jnp
from jax import lax
from jax.experimental import pallas as pl
from jax.experimental.pallas import tpu as pltpu
from jax.experimental.pallas import tpu_sc as plsc

_N = 10000
_E = 320000
_B = 64
_NC = 2
_NS = 16
_NW = _NC * _NS
_NPAD = 10240
_RPS = _NPAD // _NS
_EPW = _E // _NW
_K = 80
_NCH = _EPW // _K
_PAIRS = (_NCH - 1) // 2
_DK = 125
_DCH = _EPW // _DK
_RB = 2000
_GRID = _N // _RB
_EPS = 1e-5

_MESH = dict(core_axis_name="c", subcore_axis_name="s",
             num_cores=_NC, num_subcores=_NS)


def _make_sc_degree():
    @functools.partial(
        pl.kernel,
        out_type=jax.ShapeDtypeStruct((_NC, _NPAD, 128), jnp.float32),
        mesh=plsc.VectorSubcoreMesh(**_MESH),
        scratch_types=[
            pltpu.VMEM((_DCH, _DK), jnp.int32),
            pltpu.VMEM((_DK, 128), jnp.float32),
            pltpu.VMEM_SHARED((_NPAD, 128), jnp.float32),
            pltpu.SemaphoreType.DMA,
        ],
    )
    def deg_kernel(dst_hbm, ones_hbm, zero_hbm, out_hbm, didx, ones_v, acc_sh,
                   ssem):
        c = lax.axis_index("c")
        s = lax.axis_index("s")
        wid = s * _NC + c
        r0 = s * _RPS
        pltpu.sync_copy(dst_hbm.at[pl.ds(wid * _DCH, _DCH)], didx)
        pltpu.sync_copy(zero_hbm, acc_sh.at[pl.ds(r0, _RPS)])
        pltpu.sync_copy(ones_hbm, ones_v)
        plsc.subcore_barrier()

        pltpu.async_copy(ones_v, acc_sh.at[didx.at[0]], ssem, add=True)

        def step(i, carry):
            pltpu.async_copy(ones_v, acc_sh.at[didx.at[i + 1]], ssem, add=True)
            pltpu.make_async_copy(ones_v, acc_sh.at[didx.at[0]], ssem).wait()
            return carry

        lax.fori_loop(0, _DCH - 1, step, 0)
        pltpu.make_async_copy(ones_v, acc_sh.at[didx.at[0]], ssem).wait()
        plsc.subcore_barrier()
        pltpu.sync_copy(acc_sh.at[pl.ds(r0, _RPS)],
                        out_hbm.at[c, pl.ds(r0, _RPS)])

    return deg_kernel


def _make_sc_segsum(fc, p):
    @functools.partial(
        pl.kernel,
        out_type=jax.ShapeDtypeStruct((p * _NC, _NPAD, fc), jnp.float32),
        mesh=plsc.VectorSubcoreMesh(**_MESH),
        scratch_types=[
            pltpu.VMEM((_K,), jnp.int32),
            pltpu.VMEM((_K,), jnp.int32),
            pltpu.VMEM((_K,), jnp.int32),
            pltpu.VMEM((_K,), jnp.int32),
            pltpu.VMEM((_K, fc), jnp.float32),
            pltpu.VMEM((_K, fc), jnp.float32),
            pltpu.VMEM_SHARED((_NPAD, fc), jnp.float32),
            pltpu.SemaphoreType.DMA,
            pltpu.SemaphoreType.DMA,
            pltpu.SemaphoreType.DMA,
            pltpu.SemaphoreType.DMA,
            pltpu.SemaphoreType.DMA,
            pltpu.SemaphoreType.DMA,
        ],
    )
    def seg_kernel(*refs):
        hps = refs[:p]
        src_hbm, dst_hbm, zero_hbm, out_hbm = refs[p:p + 4]
        (sx0, dx0, sx1, dx1, rw0, rw1, acc_sh,
         gi0, gi1, sc0, sc1, il0, il1) = refs[p + 4:]
        c = lax.axis_index("c")
        s = lax.axis_index("s")
        wid = s * _NC + c
        r0 = s * _RPS
        base = wid * _EPW

        def iload(j, sx, dx, sem):
            off = pl.multiple_of(base + j * _K, 8)
            pltpu.async_copy(src_hbm.at[pl.ds(off, _K)], sx, sem)
            pltpu.async_copy(dst_hbm.at[pl.ds(off, _K)], dx, sem)

        def iwait(sx, dx, sem):
            pltpu.make_async_copy(src_hbm.at[pl.ds(0, _K)], sx, sem).wait()
            pltpu.make_async_copy(dst_hbm.at[pl.ds(0, _K)], dx, sem).wait()

        for pp in range(p):
            hp = hps[pp]

            def gwait(rw, sem):
                pltpu.make_async_copy(hp.at[sx0], rw, sem).wait()

            def swait(rw, sem):
                pltpu.make_async_copy(rw, acc_sh.at[dx0], sem).wait()

            pltpu.sync_copy(zero_hbm, acc_sh.at[pl.ds(r0, _RPS)])
            plsc.subcore_barrier()

            iload(0, sx0, dx0, il0)
            iwait(sx0, dx0, il0)
            pltpu.async_copy(hp.at[sx0], rw0, gi0).wait()
            pltpu.async_copy(rw0, acc_sh.at[dx0], sc0, add=True)
            iload(1, sx1, dx1, il1)

            def pair(i, carry):
                a = 2 * i + 1
                iwait(sx1, dx1, il1)
                pltpu.async_copy(hp.at[sx1], rw1, gi1)
                swait(rw0, sc0)
                iload(a + 1, sx0, dx0, il0)
                gwait(rw1, gi1)
                pltpu.async_copy(rw1, acc_sh.at[dx1], sc1, add=True)
                iwait(sx0, dx0, il0)
                pltpu.async_copy(hp.at[sx0], rw0, gi0)
                gwait(rw0, gi0)
                swait(rw1, sc1)
                iload(jnp.minimum(a + 2, _NCH - 1), sx1, dx1, il1)
                pltpu.async_copy(rw0, acc_sh.at[dx0], sc0, add=True)
                return carry

            lax.fori_loop(0, _PAIRS, pair, 0)
            iwait(sx1, dx1, il1)
            swait(rw0, sc0)
            plsc.subcore_barrier()
            pltpu.sync_copy(acc_sh.at[pl.ds(r0, _RPS)],
                            out_hbm.at[pp * _NC + c, pl.ds(r0, _RPS)])
            if pp + 1 < p:
                plsc.subcore_barrier()

    return seg_kernel


def _tc_mm1(x, w1, degp):
    def body(x_ref, w_ref, deg_ref, hlo_ref, hhi_ref, dinv_ref):
        deg = 1.0 + deg_ref[0][:, 0:1] + deg_ref[1][:, 0:1]
        dinv = lax.rsqrt(deg)
        h = jnp.dot(x_ref[...], w_ref[...], preferred_element_type=jnp.float32)
        hp = h * dinv
        hlo_ref[...] = hp[:, :128]
        hhi_ref[...] = hp[:, 128:]
        dinv_ref[...] = dinv

    return pl.pallas_call(
        body,
        grid=(_GRID,),
        in_specs=[
            pl.BlockSpec((_RB, 128), lambda i: (i, 0)),
            pl.BlockSpec((128, 256), lambda i: (0, 0)),
            pl.BlockSpec((_NC, _RB, 128), lambda i: (0, i, 0)),
        ],
        out_specs=[
            pl.BlockSpec((_RB, 128), lambda i: (i, 0)),
            pl.BlockSpec((_RB, 128), lambda i: (i, 0)),
            pl.BlockSpec((_RB, 1), lambda i: (i, 0)),
        ],
        out_shape=[
            jax.ShapeDtypeStruct((_N, 128), jnp.float32),
            jax.ShapeDtypeStruct((_N, 128), jnp.float32),
            jax.ShapeDtypeStruct((_N, 1), jnp.float32),
        ],
    )(x, w1, degp)


def _tc_stats(parts, hps, dinv, b, f, p, wc=128):
    fc = f // p

    def body(*refs):
        parts_ref = refs[0]
        hp_refs = refs[1:1 + p]
        dinv_ref, b_ref = refs[1 + p:3 + p]
        op_ref, mu_ref, var_ref = refs[3 + p:6 + p]
        s_acc, ss_acc = refs[6 + p:]
        i = pl.program_id(0)
        dinv = dinv_ref[...]
        cols = []
        for pp in range(p):
            tot = parts_ref[2 * pp] + parts_ref[2 * pp + 1] + hp_refs[pp][...]
            cols.append(dinv * tot[:, :fc])
        outb = (jnp.concatenate(cols, axis=1) if p > 1 else cols[0]) + b_ref[...]
        op_ref[...] = outb

        @pl.when(i == 0)
        def _():
            s_acc[...] = jnp.zeros_like(s_acc)
            ss_acc[...] = jnp.zeros_like(ss_acc)

        s_acc[...] += jnp.sum(outb, axis=0, keepdims=True)
        ss_acc[...] += jnp.sum(outb * outb, axis=0, keepdims=True)

        @pl.when(i == _GRID - 1)
        def _():
            mu = s_acc[...] * (1.0 / _N)
            mu_ref[...] = mu
            var_ref[...] = ss_acc[...] * (1.0 / _N) - mu * mu

    return pl.pallas_call(
        body,
        grid=(_GRID,),
        in_specs=(
            [pl.BlockSpec((p * _NC, _RB, wc), lambda i: (0, i, 0))]
            + [pl.BlockSpec((_RB, wc), lambda i: (i, 0))] * p
            + [pl.BlockSpec((_RB, 1), lambda i: (i, 0)),
               pl.BlockSpec((1, f), lambda i: (0, 0))]
        ),
        out_specs=[
            pl.BlockSpec((_RB, f), lambda i: (i, 0)),
            pl.BlockSpec((1, f), lambda i: (0, 0)),
            pl.BlockSpec((1, f), lambda i: (0, 0)),
        ],
        out_shape=[
            jax.ShapeDtypeStruct((_N, f), jnp.float32),
            jax.ShapeDtypeStruct((1, f), jnp.float32),
            jax.ShapeDtypeStruct((1, f), jnp.float32),
        ],
        scratch_shapes=[
            pltpu.VMEM((1, f), jnp.float32),
            pltpu.VMEM((1, f), jnp.float32),
        ],
    )(parts, *hps, dinv, b)


def _tc_bnmm(op, mu, var, g, be, w, dinv, f, fn):
    def body(op_ref, mu_ref, var_ref, g_ref, be_ref, w_ref, dinv_ref, out_ref):
        hn = (op_ref[...] - mu_ref[...]) * lax.rsqrt(var_ref[...] + _EPS)
        hn = jnp.maximum(hn * g_ref[...] + be_ref[...], 0.0)
        out_ref[...] = jnp.dot(
            hn, w_ref[...], preferred_element_type=jnp.float32) * dinv_ref[...]

    return pl.pallas_call(
        body,
        grid=(_GRID,),
        in_specs=[
            pl.BlockSpec((_RB, f), lambda i: (i, 0)),
            pl.BlockSpec((1, f), lambda i: (0, 0)),
            pl.BlockSpec((1, f), lambda i: (0, 0)),
            pl.BlockSpec((1, f), lambda i: (0, 0)),
            pl.BlockSpec((1, f), lambda i: (0, 0)),
            pl.BlockSpec((f, fn), lambda i: (0, 0)),
            pl.BlockSpec((_RB, 1), lambda i: (i, 0)),
        ],
        out_specs=pl.BlockSpec((_RB, fn), lambda i: (i, 0)),
        out_shape=jax.ShapeDtypeStruct((_N, fn), jnp.float32),
    )(op, mu, var, g, be, w, dinv)


def _tc_tail(op3, mu3, var3, g3, be3, lng, lnb, att, batch2d,
             mw1, mb1, mw2, mb2, mw3, mb3):
    def body(op_ref, mu_ref, var_ref, g_ref, be_ref, lg_ref, lb_ref, att_ref,
             bt_ref, w1_ref, c1_ref, w2_ref, c2_ref, w3_ref, c3_ref, out_ref):
        h = (op_ref[...] - mu_ref[...]) * lax.rsqrt(var_ref[...] + _EPS)
        h = jnp.maximum(h * g_ref[...] + be_ref[...], 0.0)
        rmu = jnp.mean(h, axis=1, keepdims=True)
        rvar = jnp.mean((h - rmu) ** 2, axis=1, keepdims=True)
        hln = (h - rmu) * lax.rsqrt(rvar + _EPS) * lg_ref[...] + lb_ref[...]
        t = jnp.tanh(jnp.dot(hln, att_ref[...],
                             preferred_element_type=jnp.float32))
        m = jnp.max(t, axis=0, keepdims=True)
        e = jnp.exp(t - m)
        aw = e / jnp.sum(e, axis=0, keepdims=True)
        hw = hln * aw
        ids = lax.broadcasted_iota(jnp.int32, (_B, 1), 0)
        oh = (bt_ref[...] == ids).astype(jnp.float32)
        sums = jnp.dot(oh, hw, preferred_element_type=jnp.float32)
        cnt = jnp.sum(oh, axis=1, keepdims=True)
        pooled = sums / jnp.maximum(cnt, 1.0)
        z = jnp.maximum(jnp.dot(pooled, w1_ref[...],
                                preferred_element_type=jnp.float32)
                        + c1_ref[...], 0.0)
        z = jnp.maximum(jnp.dot(z, w2_ref[...],
                                preferred_element_type=jnp.float32)
                        + c2_ref[...], 0.0)
        out_ref[...] = jnp.dot(z, w3_ref[...],
                               preferred_element_type=jnp.float32) + c3_ref[...]

    return pl.pallas_call(
        body,
        out_shape=jax.ShapeDtypeStruct((_B, 2), jnp.float32),
    )(op3, mu3, var3, g3, be3, lng, lnb, att, batch2d,
      mw1, mb1, mw2, mb2, mw3, mb3)


_deg_call = _make_sc_degree()
_seg128x2 = _make_sc_segsum(128, 2)
_seg128x1 = _make_sc_segsum(128, 1)


def kernel(x, edge_index, batch, W1, b1, g1, be1, W2, b2, g2, be2,
           W3, b3, g3, be3, ln_g, ln_b, att, mw1, mb1, mw2, mb2, mw3, mb3):
    src = edge_index[0]
    dst = edge_index[1]
    dst2d = dst.reshape(_E // _DK, _DK)
    batch2d = batch.reshape(1, _N)
    ones128 = jnp.ones((_DK, 128), jnp.float32)
    z128 = jnp.zeros((_RPS, 128), jnp.float32)
    w3p = jnp.concatenate([W3, jnp.zeros((128, 64), jnp.float32)], axis=1)
    row = lambda v: v.reshape(1, -1)

    degp = _deg_call(dst2d, ones128, z128)
    hlo, hhi, dinv = _tc_mm1(x, W1, degp)

    parts1 = _seg128x2(hlo, hhi, src, dst, z128)
    op1, mu1, var1 = _tc_stats(parts1, [hlo, hhi], dinv, row(b1), 256, 2)
    hp2 = _tc_bnmm(op1, mu1, var1, row(g1), row(be1), W2, dinv, 256, 128)

    parts2 = _seg128x1(hp2, src, dst, z128)
    op2, mu2, var2 = _tc_stats(parts2, [hp2], dinv, row(b2), 128, 1)
    hp3 = _tc_bnmm(op2, mu2, var2, row(g2), row(be2), w3p, dinv, 128, 128)

    parts3 = _seg128x1(hp3, src, dst, z128)
    op3, mu3, var3 = _tc_stats(parts3, [hp3], dinv, row(b3), 64, 1)

    return _tc_tail(op3, mu3, var3, row(g3), row(be3), row(ln_g), row(ln_b),
                    att, batch2d, mw1, row(mb1), mw2, row(mb2), mw3, row(mb3))


# speedup vs baseline: 18.3081x; 1.2534x over previous
import functools

import jax
import jax.numpy as jnp
from jax import lax
from jax.experimental import pallas as pl
from jax.experimental.pallas import tpu as pltpu
from jax.experimental.pallas import tpu_sc as plsc

_N = 10000
_E = 320000
_B = 64
_NC = 2
_NS = 16
_NW = _NC * _NS
_NPAD = 10240
_RPS = _NPAD // _NS
_EPW = _E // _NW
_K = 80
_NCH = _EPW // _K
_PAIRS = (_NCH - 1) // 2
_DK = 125
_DCH = _EPW // _DK
_RB = 2000
_GRID = _N // _RB
_EPS = 1e-5

_MESH = dict(core_axis_name="c", subcore_axis_name="s",
             num_cores=_NC, num_subcores=_NS)


def _make_sc_degree():
    @functools.partial(
        pl.kernel,
        out_type=jax.ShapeDtypeStruct((_NC, _NPAD, 128), jnp.float32),
        mesh=plsc.VectorSubcoreMesh(**_MESH),
        scratch_types=[
            pltpu.VMEM((_DCH, _DK), jnp.int32),
            pltpu.VMEM((_DK, 128), jnp.float32),
            pltpu.VMEM_SHARED((_NPAD, 128), jnp.float32),
            pltpu.SemaphoreType.DMA,
        ],
    )
    def deg_kernel(dst_hbm, ones_hbm, zero_hbm, out_hbm, didx, ones_v, acc_sh,
                   ssem):
        c = lax.axis_index("c")
        s = lax.axis_index("s")
        wid = s * _NC + c
        r0 = s * _RPS
        pltpu.sync_copy(dst_hbm.at[pl.ds(wid * _DCH, _DCH)], didx)
        pltpu.sync_copy(zero_hbm, acc_sh.at[pl.ds(r0, _RPS)])
        pltpu.sync_copy(ones_hbm, ones_v)
        plsc.subcore_barrier()

        pltpu.async_copy(ones_v, acc_sh.at[didx.at[0]], ssem, add=True)

        def step(i, carry):
            pltpu.async_copy(ones_v, acc_sh.at[didx.at[i + 1]], ssem, add=True)
            pltpu.make_async_copy(ones_v, acc_sh.at[didx.at[0]], ssem).wait()
            return carry

        lax.fori_loop(0, _DCH - 1, step, 0)
        pltpu.make_async_copy(ones_v, acc_sh.at[didx.at[0]], ssem).wait()
        plsc.subcore_barrier()
        pltpu.sync_copy(acc_sh.at[pl.ds(r0, _RPS)],
                        out_hbm.at[c, pl.ds(r0, _RPS)])

    return deg_kernel


def _make_sc_segsum(fc, p):
    @functools.partial(
        pl.kernel,
        out_type=jax.ShapeDtypeStruct((p * _NC, _NPAD, fc), jnp.float32),
        mesh=plsc.VectorSubcoreMesh(**_MESH),
        scratch_types=[
            pltpu.VMEM((_K,), jnp.int32),
            pltpu.VMEM((_K,), jnp.int32),
            pltpu.VMEM((_K,), jnp.int32),
            pltpu.VMEM((_K,), jnp.int32),
            pltpu.VMEM((_K, fc), jnp.float32),
            pltpu.VMEM((_K, fc), jnp.float32),
            pltpu.VMEM_SHARED((_NPAD, fc), jnp.float32),
            pltpu.SemaphoreType.DMA,
            pltpu.SemaphoreType.DMA,
            pltpu.SemaphoreType.DMA,
            pltpu.SemaphoreType.DMA,
            pltpu.SemaphoreType.DMA,
            pltpu.SemaphoreType.DMA,
        ],
    )
    def seg_kernel(*refs):
        hps = refs[:p]
        src_hbm, dst_hbm, zero_hbm, out_hbm = refs[p:p + 4]
        (sx0, dx0, sx1, dx1, rw0, rw1, acc_sh,
         gi0, gi1, sc0, sc1, il0, il1) = refs[p + 4:]
        c = lax.axis_index("c")
        s = lax.axis_index("s")
        wid = s * _NC + c
        r0 = s * _RPS
        base = wid * _EPW

        def iload(j, sx, dx, sem):
            off = pl.multiple_of(base + j * _K, 8)
            pltpu.async_copy(src_hbm.at[pl.ds(off, _K)], sx, sem)
            pltpu.async_copy(dst_hbm.at[pl.ds(off, _K)], dx, sem)

        def iwait(sx, dx, sem):
            pltpu.make_async_copy(src_hbm.at[pl.ds(0, _K)], sx, sem).wait()
            pltpu.make_async_copy(dst_hbm.at[pl.ds(0, _K)], dx, sem).wait()

        for pp in range(p):
            hp = hps[pp]

            def gwait(rw, sem):
                pltpu.make_async_copy(hp.at[sx0], rw, sem).wait()

            def swait(rw, sem):
                pltpu.make_async_copy(rw, acc_sh.at[dx0], sem).wait()

            pltpu.sync_copy(zero_hbm, acc_sh.at[pl.ds(r0, _RPS)])
            plsc.subcore_barrier()

            iload(0, sx0, dx0, il0)
            iwait(sx0, dx0, il0)
            pltpu.async_copy(hp.at[sx0], rw0, gi0).wait()
            pltpu.async_copy(rw0, acc_sh.at[dx0], sc0, add=True)
            iload(1, sx1, dx1, il1)

            def pair(i, carry):
                a = 2 * i + 1
                iwait(sx1, dx1, il1)
                pltpu.async_copy(hp.at[sx1], rw1, gi1)
                swait(rw0, sc0)
                iload(a + 1, sx0, dx0, il0)
                gwait(rw1, gi1)
                pltpu.async_copy(rw1, acc_sh.at[dx1], sc1, add=True)
                iwait(sx0, dx0, il0)
                pltpu.async_copy(hp.at[sx0], rw0, gi0)
                gwait(rw0, gi0)
                swait(rw1, sc1)
                iload(jnp.minimum(a + 2, _NCH - 1), sx1, dx1, il1)
                pltpu.async_copy(rw0, acc_sh.at[dx0], sc0, add=True)
                return carry

            lax.fori_loop(0, _PAIRS, pair, 0)
            iwait(sx1, dx1, il1)
            swait(rw0, sc0)
            plsc.subcore_barrier()
            pltpu.sync_copy(acc_sh.at[pl.ds(r0, _RPS)],
                            out_hbm.at[pp * _NC + c, pl.ds(r0, _RPS)])
            if pp + 1 < p:
                plsc.subcore_barrier()

    return seg_kernel


def _tc_xprime(x, degp):
    def body(x_ref, deg_ref, xp_ref, dinv_ref):
        deg = 1.0 + deg_ref[0][:, 0:1] + deg_ref[1][:, 0:1]
        dinv = lax.rsqrt(deg)
        xp_ref[...] = x_ref[...] * dinv
        dinv_ref[...] = dinv

    return pl.pallas_call(
        body,
        grid=(_GRID,),
        in_specs=[
            pl.BlockSpec((_RB, 128), lambda i: (i, 0)),
            pl.BlockSpec((_NC, _RB, 128), lambda i: (0, i, 0)),
        ],
        out_specs=[
            pl.BlockSpec((_RB, 128), lambda i: (i, 0)),
            pl.BlockSpec((_RB, 1), lambda i: (i, 0)),
        ],
        out_shape=[
            jax.ShapeDtypeStruct((_N, 128), jnp.float32),
            jax.ShapeDtypeStruct((_N, 1), jnp.float32),
        ],
    )(x, degp)


def _tc_stats_mm(parts, xp, dinv, w1, b, f):
    def body(parts_ref, xp_ref, dinv_ref, w_ref, b_ref,
             op_ref, mu_ref, var_ref, s_acc, ss_acc):
        i = pl.program_id(0)
        y = parts_ref[0] + parts_ref[1] + xp_ref[...]
        outb = dinv_ref[...] * jnp.dot(
            y, w_ref[...], preferred_element_type=jnp.float32) + b_ref[...]
        op_ref[...] = outb

        @pl.when(i == 0)
        def _():
            s_acc[...] = jnp.zeros_like(s_acc)
            ss_acc[...] = jnp.zeros_like(ss_acc)

        s_acc[...] += jnp.sum(outb, axis=0, keepdims=True)
        ss_acc[...] += jnp.sum(outb * outb, axis=0, keepdims=True)

        @pl.when(i == _GRID - 1)
        def _():
            mu = s_acc[...] * (1.0 / _N)
            mu_ref[...] = mu
            var_ref[...] = ss_acc[...] * (1.0 / _N) - mu * mu

    return pl.pallas_call(
        body,
        grid=(_GRID,),
        in_specs=[
            pl.BlockSpec((_NC, _RB, 128), lambda i: (0, i, 0)),
            pl.BlockSpec((_RB, 128), lambda i: (i, 0)),
            pl.BlockSpec((_RB, 1), lambda i: (i, 0)),
            pl.BlockSpec((128, f), lambda i: (0, 0)),
            pl.BlockSpec((1, f), lambda i: (0, 0)),
        ],
        out_specs=[
            pl.BlockSpec((_RB, f), lambda i: (i, 0)),
            pl.BlockSpec((1, f), lambda i: (0, 0)),
            pl.BlockSpec((1, f), lambda i: (0, 0)),
        ],
        out_shape=[
            jax.ShapeDtypeStruct((_N, f), jnp.float32),
            jax.ShapeDtypeStruct((1, f), jnp.float32),
            jax.ShapeDtypeStruct((1, f), jnp.float32),
        ],
        scratch_shapes=[
            pltpu.VMEM((1, f), jnp.float32),
            pltpu.VMEM((1, f), jnp.float32),
        ],
    )(parts, xp, dinv, w1, b)


def _tc_stats(parts, hps, dinv, b, f, p, wc=128):
    fc = f // p

    def body(*refs):
        parts_ref = refs[0]
        hp_refs = refs[1:1 + p]
        dinv_ref, b_ref = refs[1 + p:3 + p]
        op_ref, mu_ref, var_ref = refs[3 + p:6 + p]
        s_acc, ss_acc = refs[6 + p:]
        i = pl.program_id(0)
        dinv = dinv_ref[...]
        cols = []
        for pp in range(p):
            tot = parts_ref[2 * pp] + parts_ref[2 * pp + 1] + hp_refs[pp][...]
            cols.append(dinv * tot[:, :fc])
        outb = (jnp.concatenate(cols, axis=1) if p > 1 else cols[0]) + b_ref[...]
        op_ref[...] = outb

        @pl.when(i == 0)
        def _():
            s_acc[...] = jnp.zeros_like(s_acc)
            ss_acc[...] = jnp.zeros_like(ss_acc)

        s_acc[...] += jnp.sum(outb, axis=0, keepdims=True)
        ss_acc[...] += jnp.sum(outb * outb, axis=0, keepdims=True)

        @pl.when(i == _GRID - 1)
        def _():
            mu = s_acc[...] * (1.0 / _N)
            mu_ref[...] = mu
            var_ref[...] = ss_acc[...] * (1.0 / _N) - mu * mu

    return pl.pallas_call(
        body,
        grid=(_GRID,),
        in_specs=(
            [pl.BlockSpec((p * _NC, _RB, wc), lambda i: (0, i, 0))]
            + [pl.BlockSpec((_RB, wc), lambda i: (i, 0))] * p
            + [pl.BlockSpec((_RB, 1), lambda i: (i, 0)),
               pl.BlockSpec((1, f), lambda i: (0, 0))]
        ),
        out_specs=[
            pl.BlockSpec((_RB, f), lambda i: (i, 0)),
            pl.BlockSpec((1, f), lambda i: (0, 0)),
            pl.BlockSpec((1, f), lambda i: (0, 0)),
        ],
        out_shape=[
            jax.ShapeDtypeStruct((_N, f), jnp.float32),
            jax.ShapeDtypeStruct((1, f), jnp.float32),
            jax.ShapeDtypeStruct((1, f), jnp.float32),
        ],
        scratch_shapes=[
            pltpu.VMEM((1, f), jnp.float32),
            pltpu.VMEM((1, f), jnp.float32),
        ],
    )(parts, *hps, dinv, b)


def _tc_bnmm(op, mu, var, g, be, w, dinv, f, fn):
    def body(op_ref, mu_ref, var_ref, g_ref, be_ref, w_ref, dinv_ref, out_ref):
        hn = (op_ref[...] - mu_ref[...]) * lax.rsqrt(var_ref[...] + _EPS)
        hn = jnp.maximum(hn * g_ref[...] + be_ref[...], 0.0)
        out_ref[...] = jnp.dot(
            hn, w_ref[...], preferred_element_type=jnp.float32) * dinv_ref[...]

    return pl.pallas_call(
        body,
        grid=(_GRID,),
        in_specs=[
            pl.BlockSpec((_RB, f), lambda i: (i, 0)),
            pl.BlockSpec((1, f), lambda i: (0, 0)),
            pl.BlockSpec((1, f), lambda i: (0, 0)),
            pl.BlockSpec((1, f), lambda i: (0, 0)),
            pl.BlockSpec((1, f), lambda i: (0, 0)),
            pl.BlockSpec((f, fn), lambda i: (0, 0)),
            pl.BlockSpec((_RB, 1), lambda i: (i, 0)),
        ],
        out_specs=pl.BlockSpec((_RB, fn), lambda i: (i, 0)),
        out_shape=jax.ShapeDtypeStruct((_N, fn), jnp.float32),
    )(op, mu, var, g, be, w, dinv)


def _tc_tail(op3, mu3, var3, g3, be3, lng, lnb, att, batch2d,
             mw1, mb1, mw2, mb2, mw3, mb3):
    def body(op_ref, mu_ref, var_ref, g_ref, be_ref, lg_ref, lb_ref, att_ref,
             bt_ref, w1_ref, c1_ref, w2_ref, c2_ref, w3_ref, c3_ref, out_ref):
        h = (op_ref[...] - mu_ref[...]) * lax.rsqrt(var_ref[...] + _EPS)
        h = jnp.maximum(h * g_ref[...] + be_ref[...], 0.0)
        rmu = jnp.mean(h, axis=1, keepdims=True)
        rvar = jnp.mean((h - rmu) ** 2, axis=1, keepdims=True)
        hln = (h - rmu) * lax.rsqrt(rvar + _EPS) * lg_ref[...] + lb_ref[...]
        t = jnp.tanh(jnp.dot(hln, att_ref[...],
                             preferred_element_type=jnp.float32))
        m = jnp.max(t, axis=0, keepdims=True)
        e = jnp.exp(t - m)
        aw = e / jnp.sum(e, axis=0, keepdims=True)
        hw = hln * aw
        ids = lax.broadcasted_iota(jnp.int32, (_B, 1), 0)
        oh = (bt_ref[...] == ids).astype(jnp.float32)
        sums = jnp.dot(oh, hw, preferred_element_type=jnp.float32)
        cnt = jnp.sum(oh, axis=1, keepdims=True)
        pooled = sums / jnp.maximum(cnt, 1.0)
        z = jnp.maximum(jnp.dot(pooled, w1_ref[...],
                                preferred_element_type=jnp.float32)
                        + c1_ref[...], 0.0)
        z = jnp.maximum(jnp.dot(z, w2_ref[...],
                                preferred_element_type=jnp.float32)
                        + c2_ref[...], 0.0)
        out_ref[...] = jnp.dot(z, w3_ref[...],
                               preferred_element_type=jnp.float32) + c3_ref[...]

    return pl.pallas_call(
        body,
        out_shape=jax.ShapeDtypeStruct((_B, 2), jnp.float32),
    )(op3, mu3, var3, g3, be3, lng, lnb, att, batch2d,
      mw1, mb1, mw2, mb2, mw3, mb3)


_deg_call = _make_sc_degree()
_seg128x1 = _make_sc_segsum(128, 1)


def kernel(x, edge_index, batch, W1, b1, g1, be1, W2, b2, g2, be2,
           W3, b3, g3, be3, ln_g, ln_b, att, mw1, mb1, mw2, mb2, mw3, mb3):
    src = edge_index[0]
    dst = edge_index[1]
    dst2d = dst.reshape(_E // _DK, _DK)
    batch2d = batch.reshape(1, _N)
    ones128 = jnp.ones((_DK, 128), jnp.float32)
    z128 = jnp.zeros((_RPS, 128), jnp.float32)
    w3p = jnp.concatenate([W3, jnp.zeros((128, 64), jnp.float32)], axis=1)
    row = lambda v: v.reshape(1, -1)

    degp = _deg_call(dst2d, ones128, z128)
    xp, dinv = _tc_xprime(x, degp)

    parts1 = _seg128x1(xp, src, dst, z128)
    op1, mu1, var1 = _tc_stats_mm(parts1, xp, dinv, W1, row(b1), 256)
    hp2 = _tc_bnmm(op1, mu1, var1, row(g1), row(be1), W2, dinv, 256, 128)

    parts2 = _seg128x1(hp2, src, dst, z128)
    op2, mu2, var2 = _tc_stats(parts2, [hp2], dinv, row(b2), 128, 1)
    hp3 = _tc_bnmm(op2, mu2, var2, row(g2), row(be2), w3p, dinv, 128, 128)

    parts3 = _seg128x1(hp3, src, dst, z128)
    op3, mu3, var3 = _tc_stats(parts3, [hp3], dinv, row(b3), 64, 1)

    return _tc_tail(op3, mu3, var3, row(g3), row(be3), row(ln_g), row(ln_b),
                    att, batch2d, mw1, row(mb1), mw2, row(mb2), mw3, row(mb3))
